# baseline (speedup 1.0000x reference)
import jax
import jax.numpy as jnp
from jax.experimental import pallas as pl


_BM = 512


def _router_body(x_ref, wt_ref, out_ref):
    out_ref[...] = jnp.dot(
        x_ref[...], wt_ref[...], preferred_element_type=jnp.float32
    )


def kernel(x, W):
    m, k = x.shape
    e = W.shape[0]
    wt = W.T
    grid = (m // _BM,)
    return pl.pallas_call(
        _router_body,
        grid=grid,
        in_specs=[
            pl.BlockSpec((_BM, k), lambda i: (i, 0)),
            pl.BlockSpec((k, e), lambda i: (0, 0)),
        ],
        out_specs=pl.BlockSpec((_BM, e), lambda i: (i, 0)),
        out_shape=jax.ShapeDtypeStruct((m, e), jnp.float32),
    )(x, wt)


# speedup vs baseline: 1.2058x; 1.2058x over previous
import jax
import jax.numpy as jnp
from jax.experimental import pallas as pl


_BM = 1024


def _router_body(x_ref, w_ref, out_ref):
    out_ref[...] = jax.lax.dot_general(
        x_ref[...],
        w_ref[...],
        dimension_numbers=(((1,), (1,)), ((), ())),
        preferred_element_type=jnp.float32,
    )


def kernel(x, W):
    m, k = x.shape
    e = W.shape[0]
    grid = (m // _BM,)
    return pl.pallas_call(
        _router_body,
        grid=grid,
        in_specs=[
            pl.BlockSpec((_BM, k), lambda i: (i, 0)),
            pl.BlockSpec((e, k), lambda i: (0, 0)),
        ],
        out_specs=pl.BlockSpec((_BM, e), lambda i: (i, 0)),
        out_shape=jax.ShapeDtypeStruct((m, e), jnp.float32),
    )(x, W)
